# baseline (speedup 1.0000x reference)
import functools

import jax
import jax.numpy as jnp
from jax import lax
from jax.experimental import pallas as pl
from jax.experimental.pallas import tpu as pltpu
from jax.experimental.pallas import tpu_sc as plsc

NC = 2
NS = 16
NW = NC * NS
N = 65536
D = 512
H = D // 2
NB = 50
NBP = 64
ROWS_PER_W = N // NW
GC = 32
NCHS = ROWS_PER_W // GC
MAXSEG = 128
MOM = 0.9

_mesh = plsc.VectorSubcoreMesh(core_axis_name="c", subcore_axis_name="s",
                               num_cores=NC, num_subcores=NS)


def _sc_body(feat, lbl, psum, psq, pcnt, lblbuf, idxorder, gbuf, acc_s,
             acc_q, cntv, cnt_sm, cnt4_sm, fill4_sm, seg_sm, chcnt_sm,
             packed_sm, fsem):
    c = lax.axis_index("c")
    s = lax.axis_index("s")
    wid = s * NC + c
    base0 = wid * ROWS_PER_W
    lane = jnp.arange(16, dtype=jnp.int32)

    pltpu.sync_copy(lbl.at[pl.ds(base0, ROWS_PER_W)], lblbuf)

    with jax.named_scope("p1_sort"):
        def zcnt(b, carry):
            cnt4_sm[b] = 0
            return carry

        lax.fori_loop(0, 4 * NBP, zcnt, 0)

        def zch(ch, carry):
            chcnt_sm[ch] = 0
            return carry

        lax.fori_loop(0, NCHS, zch, 0)

        def hist(g, carry):
            lv = lblbuf[pl.ds(g * 16, 16)]
            for l in range(16):
                lb = lv[l] + (l & 3) * NBP
                cnt4_sm[lb] = cnt4_sm[lb] + 1
            return carry

        lax.fori_loop(0, ROWS_PER_W // 16, hist, 0)

        def bbuild(b, st):
            si, acc = st
            c0 = cnt4_sm[b]
            c1 = cnt4_sm[NBP + b]
            c2 = cnt4_sm[2 * NBP + b]
            c3 = cnt4_sm[3 * NBP + b]
            k = c0 + c1 + c2 + c3
            cnt_sm[b] = k
            fill4_sm[b] = acc
            fill4_sm[NBP + b] = acc + c0
            fill4_sm[2 * NBP + b] = acc + c0 + c1
            fill4_sm[3 * NBP + b] = acc + c0 + c1 + c2
            e0 = acc
            e1 = acc + k
            ch0 = lax.div(e0, GC)
            npieces = lax.div(e1 + (GC - 1), GC) - ch0

            def inner(j, si2, b=b, e0=e0, e1=e1, ch0=ch0):
                pstart = jnp.maximum(e0, (ch0 + j) * GC)
                pend = jnp.minimum(e1, (ch0 + j + 1) * GC)
                off = pstart - (ch0 + j) * GC
                ln = pend - pstart
                first = jnp.where(j == 0, 1, 0)
                seg_sm[si2] = (b + lax.shift_left(off, 6)
                               + lax.shift_left(ln, 12)
                               + lax.shift_left(first, 18))
                chcnt_sm[ch0 + j] = chcnt_sm[ch0 + j] + 1
                return si2 + 1

            si = lax.fori_loop(0, npieces, inner, si)
            return (si, e1)

        lax.fori_loop(0, NB, bbuild, (jnp.int32(0), jnp.int32(0)))

        def zpack(w, carry):
            packed_sm[w] = 0
            return carry

        lax.fori_loop(0, ROWS_PER_W // 2, zpack, 0)

        def pack(g, carry):
            lv = lblbuf[pl.ds(g * 16, 16)]
            for l in range(16):
                lb = lv[l] + (l & 3) * NBP
                p = fill4_sm[lb]
                fill4_sm[lb] = p + 1
                w = lax.shift_right_logical(p, 1)
                sh = lax.bitwise_and(p, 1) * 16
                packed_sm[w] = lax.bitwise_or(
                    packed_sm[w], lax.shift_left(g * 16 + l, sh))
            return carry

        lax.fori_loop(0, ROWS_PER_W // 16, pack, 0)

        def unpack(g, carry):
            vec = jnp.zeros((16,), jnp.int32)
            base_w = g * 8
            for lw in range(8):
                wv = packed_sm[base_w + lw]
                lo = lax.bitwise_and(wv, 0xFFFF)
                hi = lax.shift_right_logical(wv, 16)
                vec = jnp.where(lane == 2 * lw, lo, vec)
                vec = jnp.where(lane == 2 * lw + 1, hi, vec)
            idxorder[pl.ds(g * 16, 16)] = vec + base0
            return carry

        lax.fori_loop(0, ROWS_PER_W // 16, unpack, 0)

    with jax.named_scope("p2_gather_acc"):
        def fetch(ci):
            slot = lax.rem(ci, 2)
            return pltpu.async_copy(
                feat.at[idxorder.at[pl.ds(ci * GC, GC)]],
                gbuf.at[pl.ds(slot * GC, GC)],
                fsem.at[slot],
            )

        fetch(0)

        zv = jnp.zeros((16,), jnp.float32)

        def zempty(b, carry):
            @pl.when(cnt_sm[b] == 0)
            def _():
                for j in range(D // 16):
                    acc_s[b, pl.ds(j * 16, 16)] = zv
                    acc_q[b, pl.ds(j * 16, 16)] = zv
            return carry

        lax.fori_loop(0, NB, zempty, 0)

        def chunk(ch, si0):
            slot = lax.rem(ch, 2)
            pltpu.make_async_copy(
                feat.at[idxorder.at[pl.ds(ch * GC, GC)]],
                gbuf.at[pl.ds(slot * GC, GC)],
                fsem.at[slot],
            ).wait()

            @pl.when(ch + 1 < NCHS)
            def _():
                fetch(ch + 1)

            row0 = slot * GC
            nseg = chcnt_sm[ch]

            def seg(t, carry):
                si = si0 + t
                sw = seg_sm[si]
                b = lax.bitwise_and(sw, 63)
                off = lax.bitwise_and(lax.shift_right_logical(sw, 6), 63)
                ln = lax.bitwise_and(lax.shift_right_logical(sw, 12), 63)
                for h in range(2):
                    init = tuple(
                        jnp.zeros((16,), jnp.float32) for _ in range(32))

                    def rowacc(ri, regs, h=h):
                        ss, qq = [], []
                        for j in range(16):
                            v = gbuf[row0 + off + ri,
                                     pl.ds(h * H + j * 16, 16)]
                            ss.append(regs[j] + v)
                            qq.append(regs[16 + j] + v * v)
                        return tuple(ss) + tuple(qq)

                    regs = lax.fori_loop(0, ln, rowacc, init)
                    first = lax.shift_right_logical(sw, 18)

                    @pl.when(first == 1)
                    def _(regs=regs, b=b, h=h):
                        for j in range(16):
                            acc_s[b, pl.ds(h * H + j * 16, 16)] = regs[j]
                            acc_q[b, pl.ds(h * H + j * 16, 16)] = regs[16 + j]

                    @pl.when(first == 0)
                    def _(regs=regs, b=b, h=h):
                        for j in range(16):
                            plsc.addupdate(
                                acc_s.at[b, pl.ds(h * H + j * 16, 16)],
                                regs[j])
                            plsc.addupdate(
                                acc_q.at[b, pl.ds(h * H + j * 16, 16)],
                                regs[16 + j])
                return carry

            lax.fori_loop(0, nseg, seg, 0)
            return si0 + nseg

        lax.fori_loop(0, NCHS, chunk, jnp.int32(0))

    for b in range(NBP):
        vecb = jnp.where(lane == 0, cnt_sm[b], 0).astype(jnp.float32)
        cntv[b, :] = vecb
    pltpu.sync_copy(acc_s, psum.at[wid])
    pltpu.sync_copy(acc_q, psq.at[wid])
    pltpu.sync_copy(cntv, pcnt.at[wid])


_sc_call = functools.partial(
    pl.kernel,
    out_type=(
        jax.ShapeDtypeStruct((NW, NB, D), jnp.float32),
        jax.ShapeDtypeStruct((NW, NB, D), jnp.float32),
        jax.ShapeDtypeStruct((NW, NBP, 16), jnp.float32),
    ),
    mesh=_mesh,
    scratch_types=[
        pltpu.VMEM((ROWS_PER_W,), jnp.int32),
        pltpu.VMEM((ROWS_PER_W,), jnp.int32),
        pltpu.VMEM((2 * GC, D), jnp.float32),
        pltpu.VMEM((NB, D), jnp.float32),
        pltpu.VMEM((NB, D), jnp.float32),
        pltpu.VMEM((NBP, 16), jnp.float32),
        pltpu.SMEM((NBP,), jnp.int32),
        pltpu.SMEM((4 * NBP,), jnp.int32),
        pltpu.SMEM((4 * NBP,), jnp.int32),
        pltpu.SMEM((MAXSEG,), jnp.int32),
        pltpu.SMEM((NCHS,), jnp.int32),
        pltpu.SMEM((ROWS_PER_W // 2,), jnp.int32),
        pltpu.SemaphoreType.DMA((2,)),
    ],
)(_sc_body)


def _fin_body(ps, pq, pc, rm, rv, nst, om, ov, on):
    sx = jnp.sum(ps[...], axis=0)
    qx = jnp.sum(pq[...], axis=0)
    cnt = jnp.sum(pc[...], axis=0)[:NB, 0:1]
    safe_n = jnp.maximum(cnt, 1.0)
    mean = sx / safe_n
    denom = jnp.maximum(cnt - 1.0, 1.0)
    var_u = (qx - cnt * mean * mean) / denom
    var_b = qx / safe_n - mean * mean
    var = jnp.where(cnt > 1.0, var_u, var_b)
    present = cnt > 0.0
    om[...] = jnp.where(present, (1.0 - MOM) * mean + MOM * rm[...], rm[...])
    ov[...] = jnp.where(present, (1.0 - MOM) * var + MOM * rv[...], rv[...])
    on[...] = nst[...] + cnt


_fin_call = pl.pallas_call(
    _fin_body,
    out_shape=(
        jax.ShapeDtypeStruct((NB, D), jnp.float32),
        jax.ShapeDtypeStruct((NB, D), jnp.float32),
        jax.ShapeDtypeStruct((NB, 1), jnp.float32),
    ),
)


def kernel(features, labels, running_mean, running_var, num_samples_tracked):
    psum, psq, pcnt = _sc_call(features, labels)
    new_mean, new_var, new_num = _fin_call(
        psum, psq, pcnt, running_mean, running_var,
        num_samples_tracked.reshape(NB, 1))
    return new_mean, new_var, new_num.reshape(NB)


# speedup vs baseline: 1.0748x; 1.0748x over previous
import functools

import jax
import jax.numpy as jnp
from jax import lax
from jax.experimental import pallas as pl
from jax.experimental.pallas import tpu as pltpu
from jax.experimental.pallas import tpu_sc as plsc

NC = 2
NS = 16
NW = NC * NS
N = 65536
D = 512
H = D // 2
NB = 50
NBP = 64
ROWS_PER_W = N // NW
GC = 48
NCHS = (ROWS_PER_W + GC - 1) // GC
MAXSEG = 128
MOM = 0.9

_mesh = plsc.VectorSubcoreMesh(core_axis_name="c", subcore_axis_name="s",
                               num_cores=NC, num_subcores=NS)


def _sc_body(feat, lbl, psum, psq, pcnt, lblbuf, idxorder, gbuf, acc_s,
             acc_q, cntv, cnt_sm, cnt4_sm, fill4_sm, seg_sm, chcnt_sm,
             packed_sm, fsem):
    c = lax.axis_index("c")
    s = lax.axis_index("s")
    wid = s * NC + c
    base0 = wid * ROWS_PER_W
    lane = jnp.arange(16, dtype=jnp.int32)

    pltpu.sync_copy(lbl.at[pl.ds(base0, ROWS_PER_W)], lblbuf)

    with jax.named_scope("p1_sort"):
        def zcnt(b, carry):
            cnt4_sm[b] = 0
            return carry

        lax.fori_loop(0, 4 * NBP, zcnt, 0)

        def zch(ch, carry):
            chcnt_sm[ch] = 0
            return carry

        lax.fori_loop(0, NCHS, zch, 0)

        def hist(g, carry):
            lv = lblbuf[pl.ds(g * 16, 16)]
            for l in range(16):
                lb = lv[l] + (l & 3) * NBP
                cnt4_sm[lb] = cnt4_sm[lb] + 1
            return carry

        lax.fori_loop(0, ROWS_PER_W // 16, hist, 0)

        def bbuild(b, st):
            si, acc = st
            c0 = cnt4_sm[b]
            c1 = cnt4_sm[NBP + b]
            c2 = cnt4_sm[2 * NBP + b]
            c3 = cnt4_sm[3 * NBP + b]
            k = c0 + c1 + c2 + c3
            cnt_sm[b] = k
            fill4_sm[b] = acc
            fill4_sm[NBP + b] = acc + c0
            fill4_sm[2 * NBP + b] = acc + c0 + c1
            fill4_sm[3 * NBP + b] = acc + c0 + c1 + c2
            e0 = acc
            e1 = acc + k
            ch0 = lax.div(e0, GC)
            npieces = lax.div(e1 + (GC - 1), GC) - ch0

            def inner(j, si2, b=b, e0=e0, e1=e1, ch0=ch0):
                pstart = jnp.maximum(e0, (ch0 + j) * GC)
                pend = jnp.minimum(e1, (ch0 + j + 1) * GC)
                off = pstart - (ch0 + j) * GC
                ln = pend - pstart
                first = jnp.where(j == 0, 1, 0)
                seg_sm[si2] = (b + lax.shift_left(off, 6)
                               + lax.shift_left(ln, 12)
                               + lax.shift_left(first, 18))
                chcnt_sm[ch0 + j] = chcnt_sm[ch0 + j] + 1
                return si2 + 1

            si = lax.fori_loop(0, npieces, inner, si)
            return (si, e1)

        lax.fori_loop(0, NB, bbuild, (jnp.int32(0), jnp.int32(0)))

        def zpack(w, carry):
            packed_sm[w] = 0
            return carry

        lax.fori_loop(0, ROWS_PER_W // 2, zpack, 0)

        def pack(g, carry):
            lv = lblbuf[pl.ds(g * 16, 16)]
            for l in range(16):
                lb = lv[l] + (l & 3) * NBP
                p = fill4_sm[lb]
                fill4_sm[lb] = p + 1
                w = lax.shift_right_logical(p, 1)
                sh = lax.bitwise_and(p, 1) * 16
                packed_sm[w] = lax.bitwise_or(
                    packed_sm[w], lax.shift_left(g * 16 + l, sh))
            return carry

        lax.fori_loop(0, ROWS_PER_W // 16, pack, 0)

        def unpack(g, carry):
            vec = jnp.zeros((16,), jnp.int32)
            base_w = g * 8
            for lw in range(8):
                wv = packed_sm[base_w + lw]
                lo = lax.bitwise_and(wv, 0xFFFF)
                hi = lax.shift_right_logical(wv, 16)
                vec = jnp.where(lane == 2 * lw, lo, vec)
                vec = jnp.where(lane == 2 * lw + 1, hi, vec)
            idxorder[pl.ds(g * 16, 16)] = vec + base0
            return carry

        lax.fori_loop(0, ROWS_PER_W // 16, unpack, 0)
        idxorder[pl.ds(ROWS_PER_W, 16)] = jnp.zeros((16,), jnp.int32) + base0

    with jax.named_scope("p2_gather_acc"):
        def fetch(ci):
            slot = lax.rem(ci, 2)
            return pltpu.async_copy(
                feat.at[idxorder.at[pl.ds(ci * GC, GC)]],
                gbuf.at[pl.ds(slot * GC, GC)],
                fsem.at[slot],
            )

        fetch(0)

        zv = jnp.zeros((16,), jnp.float32)

        def zempty(b, carry):
            @pl.when(cnt_sm[b] == 0)
            def _():
                for j in range(D // 16):
                    acc_s[b, pl.ds(j * 16, 16)] = zv
                    acc_q[b, pl.ds(j * 16, 16)] = zv
            return carry

        lax.fori_loop(0, NB, zempty, 0)

        def chunk(ch, si0):
            slot = lax.rem(ch, 2)
            pltpu.make_async_copy(
                feat.at[idxorder.at[pl.ds(ch * GC, GC)]],
                gbuf.at[pl.ds(slot * GC, GC)],
                fsem.at[slot],
            ).wait()

            @pl.when(ch + 1 < NCHS)
            def _():
                fetch(ch + 1)

            row0 = slot * GC
            nseg = chcnt_sm[ch]

            def seg(t, carry):
                si = si0 + t
                sw = seg_sm[si]
                b = lax.bitwise_and(sw, 63)
                off = lax.bitwise_and(lax.shift_right_logical(sw, 6), 63)
                ln = lax.bitwise_and(lax.shift_right_logical(sw, 12), 63)
                for h in range(2):
                    init = tuple(
                        jnp.zeros((16,), jnp.float32) for _ in range(32))

                    def rowacc(ri, regs, h=h):
                        ss, qq = [], []
                        for j in range(16):
                            v = gbuf[row0 + off + ri,
                                     pl.ds(h * H + j * 16, 16)]
                            ss.append(regs[j] + v)
                            qq.append(regs[16 + j] + v * v)
                        return tuple(ss) + tuple(qq)

                    regs = lax.fori_loop(0, ln, rowacc, init)
                    first = lax.shift_right_logical(sw, 18)

                    @pl.when(first == 1)
                    def _(regs=regs, b=b, h=h):
                        for j in range(16):
                            acc_s[b, pl.ds(h * H + j * 16, 16)] = regs[j]
                            acc_q[b, pl.ds(h * H + j * 16, 16)] = regs[16 + j]

                    @pl.when(first == 0)
                    def _(regs=regs, b=b, h=h):
                        for j in range(16):
                            plsc.addupdate(
                                acc_s.at[b, pl.ds(h * H + j * 16, 16)],
                                regs[j])
                            plsc.addupdate(
                                acc_q.at[b, pl.ds(h * H + j * 16, 16)],
                                regs[16 + j])
                return carry

            lax.fori_loop(0, nseg, seg, 0)
            return si0 + nseg

        lax.fori_loop(0, NCHS, chunk, jnp.int32(0))

    for b in range(NBP):
        vecb = jnp.where(lane == 0, cnt_sm[b], 0).astype(jnp.float32)
        cntv[b, :] = vecb
    pltpu.sync_copy(acc_s, psum.at[wid])
    pltpu.sync_copy(acc_q, psq.at[wid])
    pltpu.sync_copy(cntv, pcnt.at[wid])


_sc_call = functools.partial(
    pl.kernel,
    out_type=(
        jax.ShapeDtypeStruct((NW, NB, D), jnp.float32),
        jax.ShapeDtypeStruct((NW, NB, D), jnp.float32),
        jax.ShapeDtypeStruct((NW, NBP, 16), jnp.float32),
    ),
    mesh=_mesh,
    scratch_types=[
        pltpu.VMEM((ROWS_PER_W,), jnp.int32),
        pltpu.VMEM((NCHS * GC,), jnp.int32),
        pltpu.VMEM((2 * GC, D), jnp.float32),
        pltpu.VMEM((NB, D), jnp.float32),
        pltpu.VMEM((NB, D), jnp.float32),
        pltpu.VMEM((NBP, 16), jnp.float32),
        pltpu.SMEM((NBP,), jnp.int32),
        pltpu.SMEM((4 * NBP,), jnp.int32),
        pltpu.SMEM((4 * NBP,), jnp.int32),
        pltpu.SMEM((MAXSEG,), jnp.int32),
        pltpu.SMEM((NCHS,), jnp.int32),
        pltpu.SMEM((ROWS_PER_W // 2,), jnp.int32),
        pltpu.SemaphoreType.DMA((2,)),
    ],
)(_sc_body)


def _fin_body(ps, pq, pc, rm, rv, nst, om, ov, on):
    sx = jnp.sum(ps[...], axis=0)
    qx = jnp.sum(pq[...], axis=0)
    cnt = jnp.sum(pc[...], axis=0)[:NB, 0:1]
    safe_n = jnp.maximum(cnt, 1.0)
    mean = sx / safe_n
    denom = jnp.maximum(cnt - 1.0, 1.0)
    var_u = (qx - cnt * mean * mean) / denom
    var_b = qx / safe_n - mean * mean
    var = jnp.where(cnt > 1.0, var_u, var_b)
    present = cnt > 0.0
    om[...] = jnp.where(present, (1.0 - MOM) * mean + MOM * rm[...], rm[...])
    ov[...] = jnp.where(present, (1.0 - MOM) * var + MOM * rv[...], rv[...])
    on[...] = nst[...] + cnt


_fin_call = pl.pallas_call(
    _fin_body,
    out_shape=(
        jax.ShapeDtypeStruct((NB, D), jnp.float32),
        jax.ShapeDtypeStruct((NB, D), jnp.float32),
        jax.ShapeDtypeStruct((NB, 1), jnp.float32),
    ),
)


def kernel(features, labels, running_mean, running_var, num_samples_tracked):
    psum, psq, pcnt = _sc_call(features, labels)
    new_mean, new_var, new_num = _fin_call(
        psum, psq, pcnt, running_mean, running_var,
        num_samples_tracked.reshape(NB, 1))
    return new_mean, new_var, new_num.reshape(NB)
